# baseline (speedup 1.0000x reference)
import dataclasses
import functools

import jax
import jax.numpy as jnp
from jax import lax
from jax.experimental import pallas as pl
from jax.experimental.pallas import tpu as pltpu
from jax.experimental.pallas import tpu_sc as plsc

B = 4096
A = 26
C = 64
NW = 32
SPW = B // NW
NST = 4
L = 16
FW = 256
HUGE = 2147483647


def _compiler_params():
    cp = pltpu.CompilerParams()
    fields = pltpu.CompilerParams.__dataclass_fields__
    if "needs_layout_passes" in fields:
        cp = dataclasses.replace(cp, needs_layout_passes=False)
    return cp


def _sc_gather(param_t, sample_idx, index_mat):
    n_params = param_t.shape[1]
    npad = -(-n_params // 128) * 128
    mesh = plsc.VectorSubcoreMesh(core_axis_name="c", subcore_axis_name="s")

    @functools.partial(
        pl.kernel,
        out_type=jax.ShapeDtypeStruct((B, A, C), jnp.float32),
        mesh=mesh,
        compiler_params=_compiler_params(),
        scratch_types=(
            [pltpu.VMEM((SPW,), jnp.int32),
             pltpu.VMEM((SPW, 32), jnp.int32)]
            + [pltpu.VMEM((A, C), jnp.float32) for _ in range(NST)]
            + [pltpu.VMEM((C, FW), jnp.float32) for _ in range(NST)]
            + [pltpu.SMEM((2 * NST,), jnp.int32),
               pltpu.SemaphoreType.DMA]
            + [pltpu.SemaphoreType.DMA for _ in range(2 * NST)]
        ),
    )
    def k(pt_hbm, sidx_hbm, imat_hbm, out_hbm, sidx_v, imat_v, *scr):
        rows = scr[0:NST]
        fb = scr[NST:2 * NST]
        st_s = scr[2 * NST]
        isem = scr[2 * NST + 1]
        fsem = scr[2 * NST + 2:3 * NST + 2]
        osem = scr[3 * NST + 2:4 * NST + 2]

        wid = lax.axis_index("s") * 2 + lax.axis_index("c")
        sbase = wid * SPW

        pltpu.sync_copy(sidx_hbm.at[pl.ds(sbase, SPW)], sidx_v)

        iota = lax.broadcasted_iota(jnp.int32, (L,), 0)

        @pl.loop(0, SPW // L)
        def _(g):
            sv = sidx_v[pl.ds(g * L, L)]
            for j in range(L):
                sid = jnp.max(jnp.where(iota == j, sv, jnp.int32(0)))
                pltpu.async_copy(imat_hbm.at[sid],
                                 imat_v.at[g * L + j, pl.ds(0, A)], isem)

        @pl.loop(0, SPW)
        def _(j):
            pltpu.make_async_copy(
                imat_hbm.at[0], imat_v.at[0, pl.ds(0, A)], isem).wait()

        zeros = jnp.zeros((L,), jnp.float32)

        def issue(st, s):
            iv1 = imat_v[s, pl.ds(0, L)]
            iv2 = imat_v[s, pl.ds(L, L)]
            m1 = jnp.where(iv1 >= 0, iv1, jnp.int32(HUGE))
            m2 = jnp.where((iv2 >= 0) & (iota < A - L), iv2, jnp.int32(HUGE))
            rmin = jnp.minimum(jnp.min(m1), jnp.min(m2))
            first = jnp.minimum(rmin, jnp.int32(n_params - 1))
            s128 = pl.multiple_of((first >> 7) << 7, 128)
            st_s[st] = s128
            pltpu.async_copy(
                pt_hbm.at[pl.ds(0, C), pl.ds(s128, 128)],
                fb[st].at[pl.ds(0, C), pl.ds(0, 128)], fsem[st])
            ext = jnp.logical_and(first + (A - 1) >= s128 + 128,
                                  s128 + 256 <= npad)
            st_s[NST + st] = jnp.where(ext, jnp.int32(2), jnp.int32(1))

            @pl.when(ext)
            def _():
                pltpu.async_copy(
                    pt_hbm.at[pl.ds(0, C),
                              pl.ds(pl.multiple_of(s128 + 128, 128), 128)],
                    fb[st].at[pl.ds(0, C), pl.ds(128, 128)], fsem[st])

        def place_out(st, s):
            def _dr(_, carry):
                pltpu.make_async_copy(
                    pt_hbm.at[pl.ds(0, C), pl.ds(0, 128)],
                    fb[st].at[pl.ds(0, C), pl.ds(0, 128)], fsem[st]).wait()
                return carry

            lax.fori_loop(0, st_s[NST + st], _dr, jnp.int32(0))
            start = st_s[st]
            iv1 = imat_v[s, pl.ds(0, L)]
            iv2 = imat_v[s, pl.ds(L, L)]
            for a in range(A):
                hv = iv1 if a < L else iv2
                ia = jnp.max(jnp.where(iota == a % L, hv,
                                       jnp.int32(-HUGE - 1)))

                @pl.when(ia >= 0)
                def _(ia=ia, a=a):
                    col = jnp.full((L,), ia - start, jnp.int32)
                    for q in range(C // L):
                        v = plsc.load_gather(fb[st], [q * L + iota, col])
                        rows[st][a, pl.ds(q * L, L)] = v

                @pl.when(ia < 0)
                def _(a=a):
                    for q in range(C // L):
                        rows[st][a, pl.ds(q * L, L)] = zeros

            pltpu.async_copy(rows[st], out_hbm.at[sbase + s], osem[st])

        def wait_out(st):
            pltpu.make_async_copy(rows[st], out_hbm.at[0], osem[st]).wait()

        for st in range(NST):
            issue(st, jnp.int32(st))

        @pl.loop(0, SPW, step=NST)
        def _(t):
            for st in range(NST):
                s = t + st

                @pl.when(s >= NST)
                def _(st=st):
                    wait_out(st)

                place_out(st, s)

                @pl.when(s + NST < SPW)
                def _(st=st, s=s):
                    issue(st, s + NST)

        for st in range(NST):
            wait_out(st)

    return k(param_t, sample_idx, index_mat)


def kernel(param, sample_idx, index_mat):
    return _sc_gather(param.T, sample_idx.astype(jnp.int32),
                      index_mat.astype(jnp.int32))


# speedup vs baseline: 1.0193x; 1.0193x over previous
import dataclasses
import functools

import jax
import jax.numpy as jnp
from jax import lax
from jax.experimental import pallas as pl
from jax.experimental.pallas import tpu as pltpu
from jax.experimental.pallas import tpu_sc as plsc

B = 4096
A = 26
C = 64
NW = 32
SPW = B // NW
NST = 4
L = 16
FW = 256
HUGE = 2147483647


def _compiler_params():
    cp = pltpu.CompilerParams()
    fields = pltpu.CompilerParams.__dataclass_fields__
    if "needs_layout_passes" in fields:
        cp = dataclasses.replace(cp, needs_layout_passes=False)
    return cp


def _sc_gather(param_t, sample_idx, index_mat):
    n_params = param_t.shape[1]
    npad = -(-n_params // 128) * 128
    mesh = plsc.VectorSubcoreMesh(core_axis_name="c", subcore_axis_name="s")

    @functools.partial(
        pl.kernel,
        out_type=jax.ShapeDtypeStruct((B, A, C), jnp.float32),
        mesh=mesh,
        compiler_params=_compiler_params(),
        scratch_types=(
            [pltpu.VMEM((SPW,), jnp.int32),
             pltpu.VMEM((SPW, 32), jnp.int32)]
            + [pltpu.VMEM((A, C), jnp.float32) for _ in range(NST)]
            + [pltpu.VMEM((C, FW), jnp.float32) for _ in range(NST)]
            + [pltpu.SMEM((2 * NST,), jnp.int32),
               pltpu.SemaphoreType.DMA]
            + [pltpu.SemaphoreType.DMA for _ in range(2 * NST)]
        ),
    )
    def k(pt_hbm, sidx_hbm, imat_hbm, out_hbm, sidx_v, imat_v, *scr):
        rows = scr[0:NST]
        fb = scr[NST:2 * NST]
        st_s = scr[2 * NST]
        isem = scr[2 * NST + 1]
        fsem = scr[2 * NST + 2:3 * NST + 2]
        osem = scr[3 * NST + 2:4 * NST + 2]

        wid = lax.axis_index("s") * 2 + lax.axis_index("c")
        sbase = wid * SPW

        pltpu.sync_copy(sidx_hbm.at[pl.ds(sbase, SPW)], sidx_v)

        iota = lax.broadcasted_iota(jnp.int32, (L,), 0)

        @pl.loop(0, SPW // L)
        def _(g):
            sv = sidx_v[pl.ds(g * L, L)]
            for j in range(L):
                sid = jnp.max(jnp.where(iota == j, sv, jnp.int32(0)))
                pltpu.async_copy(imat_hbm.at[sid],
                                 imat_v.at[g * L + j, pl.ds(0, A)], isem)

        @pl.loop(0, SPW)
        def _(j):
            pltpu.make_async_copy(
                imat_hbm.at[0], imat_v.at[0, pl.ds(0, A)], isem).wait()

        zeros = jnp.zeros((L,), jnp.float32)

        def issue(st, s):
            iv1 = imat_v[s, pl.ds(0, L)]
            iv2 = imat_v[s, pl.ds(L, L)]
            m1 = jnp.where(iv1 >= 0, iv1, jnp.int32(HUGE))
            m2 = jnp.where((iv2 >= 0) & (iota < A - L), iv2, jnp.int32(HUGE))
            rmin = jnp.minimum(jnp.min(m1), jnp.min(m2))
            first = jnp.minimum(rmin, jnp.int32(n_params - 1))
            s128 = pl.multiple_of((first >> 7) << 7, 128)
            st_s[st] = s128
            pltpu.async_copy(
                pt_hbm.at[pl.ds(0, C), pl.ds(s128, 128)],
                fb[st].at[pl.ds(0, C), pl.ds(0, 128)], fsem[st])
            ok = s128 + 256 <= npad
            s2 = jnp.where(ok, s128 + 128, s128)
            pltpu.async_copy(
                pt_hbm.at[pl.ds(0, C),
                          pl.ds(pl.multiple_of(s2, 128), 128)],
                fb[st].at[pl.ds(0, C), pl.ds(128, 128)], fsem[st])

        def place_out(st, s):
            for _ in range(2):
                pltpu.make_async_copy(
                    pt_hbm.at[pl.ds(0, C), pl.ds(0, 128)],
                    fb[st].at[pl.ds(0, C), pl.ds(0, 128)], fsem[st]).wait()
            start = st_s[st]
            iv1 = imat_v[s, pl.ds(0, L)]
            iv2 = imat_v[s, pl.ds(L, L)]
            for a in range(A):
                hv = iv1 if a < L else iv2
                ia = jnp.max(jnp.where(iota == a % L, hv,
                                       jnp.int32(-HUGE - 1)))

                @pl.when(ia >= 0)
                def _(ia=ia, a=a):
                    col = jnp.full((L,), ia - start, jnp.int32)
                    for q in range(C // L):
                        v = plsc.load_gather(fb[st], [q * L + iota, col])
                        rows[st][a, pl.ds(q * L, L)] = v

                @pl.when(ia < 0)
                def _(a=a):
                    for q in range(C // L):
                        rows[st][a, pl.ds(q * L, L)] = zeros

            pltpu.async_copy(rows[st], out_hbm.at[sbase + s], osem[st])

        def wait_out(st):
            pltpu.make_async_copy(rows[st], out_hbm.at[0], osem[st]).wait()

        for st in range(NST):
            issue(st, jnp.int32(st))

        @pl.loop(0, SPW, step=NST)
        def _(t):
            for st in range(NST):
                s = t + st

                @pl.when(s >= NST)
                def _(st=st):
                    wait_out(st)

                place_out(st, s)

                @pl.when(s + NST < SPW)
                def _(st=st, s=s):
                    issue(st, s + NST)

        for st in range(NST):
            wait_out(st)

    return k(param_t, sample_idx, index_mat)


def kernel(param, sample_idx, index_mat):
    return _sc_gather(param.T, sample_idx.astype(jnp.int32),
                      index_mat.astype(jnp.int32))


# speedup vs baseline: 1.2617x; 1.2378x over previous
import dataclasses
import functools

import jax
import jax.numpy as jnp
from jax import lax
from jax.experimental import pallas as pl
from jax.experimental.pallas import tpu as pltpu
from jax.experimental.pallas import tpu_sc as plsc

B = 4096
A = 26
C = 64
NW = 32
SPW = B // NW
SPC = 2
RPC = SPC * A
NCH = SPW // SPC
L = 16
FW = 256
HUGE = 2147483647


def _compiler_params():
    cp = pltpu.CompilerParams()
    fields = pltpu.CompilerParams.__dataclass_fields__
    if "needs_layout_passes" in fields:
        cp = dataclasses.replace(cp, needs_layout_passes=False)
    return cp


def _sc_gather(param_t, sample_idx, index_mat):
    n_params = param_t.shape[1]
    npad = -(-n_params // 128) * 128
    mesh = plsc.VectorSubcoreMesh(core_axis_name="c", subcore_axis_name="s")

    @functools.partial(
        pl.kernel,
        out_type=jax.ShapeDtypeStruct((B, A, C), jnp.float32),
        mesh=mesh,
        compiler_params=_compiler_params(),
        scratch_types=[
            pltpu.VMEM((SPW,), jnp.int32),
            pltpu.VMEM((SPW, 32), jnp.int32),
            pltpu.VMEM((RPC, C), jnp.float32),
            pltpu.VMEM((RPC, C), jnp.float32),
            pltpu.VMEM((SPC * C, FW), jnp.float32),
            pltpu.VMEM((SPC * C, FW), jnp.float32),
            pltpu.SMEM((2 * SPC + 2,), jnp.int32),
            pltpu.SemaphoreType.DMA,
            pltpu.SemaphoreType.DMA,
            pltpu.SemaphoreType.DMA,
            pltpu.SemaphoreType.DMA,
            pltpu.SemaphoreType.DMA,
        ],
    )
    def k(pt_hbm, sidx_hbm, imat_hbm, out_hbm,
          sidx_v, imat_v, rows0, rows1, fb0, fb1, st_s,
          isem, fsem0, fsem1, osem0, osem1):
        wid = lax.axis_index("s") * 2 + lax.axis_index("c")
        sbase = wid * SPW

        pltpu.sync_copy(sidx_hbm.at[pl.ds(sbase, SPW)], sidx_v)

        iota = lax.broadcasted_iota(jnp.int32, (L,), 0)

        @pl.loop(0, SPW // L)
        def _(g):
            sv = sidx_v[pl.ds(g * L, L)]
            for j in range(L):
                sid = jnp.max(jnp.where(iota == j, sv, jnp.int32(0)))
                pltpu.async_copy(imat_hbm.at[sid],
                                 imat_v.at[g * L + j, pl.ds(0, A)], isem)

        @pl.loop(0, SPW)
        def _(j):
            pltpu.make_async_copy(
                imat_hbm.at[0], imat_v.at[0, pl.ds(0, A)], isem).wait()

        rows = (rows0, rows1)
        fb = (fb0, fb1)
        fsem = (fsem0, fsem1)
        osem = (osem0, osem1)
        zeros = jnp.zeros((L,), jnp.float32)

        def wait_outs(b):
            for _ in range(SPC):
                pltpu.make_async_copy(
                    rows[b].at[pl.ds(0, A)], out_hbm.at[0], osem[b]).wait()

        def issue_fetches(b, ch):
            def _one(s, ntiles):
                ss = ch * SPC + s
                iv1 = imat_v[ss, pl.ds(0, L)]
                iv2 = imat_v[ss, pl.ds(L, L)]
                m1 = jnp.where(iv1 >= 0, iv1, jnp.int32(HUGE))
                m2 = jnp.where((iv2 >= 0) & (iota < A - L), iv2,
                               jnp.int32(HUGE))
                rmin = jnp.minimum(jnp.min(m1), jnp.min(m2))
                first = jnp.minimum(rmin, jnp.int32(n_params - 1))
                s128 = pl.multiple_of((first >> 7) << 7, 128)
                st_s[b * SPC + s] = s128
                pltpu.async_copy(
                    pt_hbm.at[pl.ds(0, C), pl.ds(s128, 128)],
                    fb[b].at[pl.ds(s * C, C), pl.ds(0, 128)], fsem[b])
                ext = jnp.logical_and(first + (A - 1) >= s128 + 128,
                                      s128 + 256 <= npad)

                @pl.when(ext)
                def _():
                    pltpu.async_copy(
                        pt_hbm.at[pl.ds(0, C),
                                  pl.ds(pl.multiple_of(s128 + 128, 128), 128)],
                        fb[b].at[pl.ds(s * C, C), pl.ds(128, 128)], fsem[b])

                return ntiles + 1 + jnp.where(ext, jnp.int32(1), jnp.int32(0))

            st_s[2 * SPC + b] = lax.fori_loop(0, SPC, _one, jnp.int32(0))

        def place_and_out(b, ch):
            def _dr(_, carry):
                pltpu.make_async_copy(
                    pt_hbm.at[pl.ds(0, C), pl.ds(0, 128)],
                    fb[b].at[pl.ds(0, C), pl.ds(0, 128)], fsem[b]).wait()
                return carry

            lax.fori_loop(0, st_s[2 * SPC + b], _dr, jnp.int32(0))

            @pl.loop(0, SPC)
            def _(s):
                ss = ch * SPC + s
                start = st_s[b * SPC + s]
                iv1 = imat_v[ss, pl.ds(0, L)]
                iv2 = imat_v[ss, pl.ds(L, L)]
                for a in range(A):
                    hv = iv1 if a < L else iv2
                    ia = jnp.max(jnp.where(iota == a % L, hv,
                                           jnp.int32(-HUGE - 1)))

                    @pl.when(ia >= 0)
                    def _(ia=ia, a=a):
                        col = jnp.full((L,), ia - start, jnp.int32)
                        for q in range(C // L):
                            v = plsc.load_gather(
                                fb[b], [s * C + q * L + iota, col])
                            rows[b][s * A + a, pl.ds(q * L, L)] = v

                    @pl.when(ia < 0)
                    def _(a=a):
                        for q in range(C // L):
                            rows[b][s * A + a, pl.ds(q * L, L)] = zeros

            for s in range(SPC):
                pltpu.async_copy(rows[b].at[pl.ds(s * A, A)],
                                 out_hbm.at[sbase + ch * SPC + s], osem[b])

        issue_fetches(0, 0)
        issue_fetches(1, 1)

        @pl.loop(0, NCH, step=2)
        def _(t):
            @pl.when(t >= 2)
            def _():
                wait_outs(0)

            place_and_out(0, t)

            @pl.when(t + 2 < NCH)
            def _():
                issue_fetches(0, t + 2)

            @pl.when(t >= 2)
            def _():
                wait_outs(1)

            place_and_out(1, t + 1)

            @pl.when(t + 3 < NCH)
            def _():
                issue_fetches(1, t + 3)

        wait_outs(0)
        wait_outs(1)

    return k(param_t, sample_idx, index_mat)


def kernel(param, sample_idx, index_mat):
    return _sc_gather(param.T, sample_idx.astype(jnp.int32),
                      index_mat.astype(jnp.int32))
